# baseline (speedup 1.0000x reference)
import functools

import jax
import jax.numpy as jnp
from jax import lax
from jax.experimental import pallas as pl
from jax.experimental.pallas import tpu as pltpu
from jax.experimental.pallas import tpu_sc as plsc

CHUNK = 32
NBUF = 8


@functools.partial(
    jax.jit, static_argnames=("n", "d", "nc", "ns", "nchunks", "n_acc"))
def _sc_aggregate(h, pk3, *, n, d, nc, ns, nchunks, n_acc):
    zpart = 8 * (n_acc // (8 * ns))
    zlast = n_acc - zpart * (ns - 1)
    opart = 8 * (n // (8 * ns))
    olast = n - opart * (ns - 1)
    mesh = plsc.VectorSubcoreMesh(core_axis_name="c", subcore_axis_name="s")

    @functools.partial(
        pl.kernel,
        out_type=jax.ShapeDtypeStruct((nc, n, d), jnp.float32),
        mesh=mesh,
        scratch_types=[
            pltpu.VMEM((nchunks // 4, 4 * CHUNK), jnp.int32),
            *([pltpu.VMEM((1, CHUNK), jnp.int32)] * (2 * NBUF)),
            *([pltpu.VMEM((CHUNK, d), jnp.float32)] * NBUF),
            pltpu.VMEM_SHARED((n_acc, d), jnp.float32),
            *([pltpu.SemaphoreType.DMA] * NBUF),
        ],
    )
    def k(h_hbm, pk_hbm, out_hbm, pk_v, *rest):
        sus = rest[0:2 * NBUF:2]
        dus = rest[1:2 * NBUF:2]
        rows = rest[2 * NBUF:3 * NBUF]
        acc_sh = rest[3 * NBUF]
        sems = rest[3 * NBUF + 1:]
        rows_v = rows[0]
        cid = lax.axis_index("c")
        sid = lax.axis_index("s")
        wid = cid * ns + sid

        def zrow(i, _):
            for k8 in range(d // 16):
                rows_v[i, pl.ds(k8 * 16, 16)] = jnp.zeros((16,), jnp.float32)
            return 0

        lax.fori_loop(0, CHUNK, zrow, 0)

        def zero_rows(nrows):
            base = sid * zpart
            full, rem = nrows // CHUNK, nrows % CHUNK
            for z in range(full):
                pltpu.sync_copy(rows_v, acc_sh.at[pl.ds(base + z * CHUNK, CHUNK)])
            if rem:
                pltpu.sync_copy(rows_v.at[pl.ds(0, rem)],
                                acc_sh.at[pl.ds(base + full * CHUNK, rem)])

        pl.when(sid < ns - 1)(lambda: zero_rows(zpart))
        pl.when(sid == ns - 1)(lambda: zero_rows(zlast))
        plsc.subcore_barrier()

        pltpu.sync_copy(pk_hbm.at[wid], pk_v)

        hoff = (cid * 4 + lax.div(sid, 4)) * n

        def unpack(j, su, du):
            half = (j % 4) * CHUNK
            for k8 in range(CHUNK // 16):
                w = pk_v[j // 4, pl.ds(half + k8 * 16, 16)]
                su[0, pl.ds(k8 * 16, 16)] = (w & 0xFFFF) + hoff
                du[0, pl.ds(k8 * 16, 16)] = lax.shift_right_logical(w, 16)

        drains = [pltpu.make_async_copy(h_hbm.at[sus[b].at[0]], rows[b], sems[b])
                  for b in range(NBUF)]
        for b in range(NBUF):
            unpack(b, sus[b], dus[b])
            pltpu.async_copy(h_hbm.at[sus[b].at[0]], rows[b], sems[b])

        def body(g, _):
            j = NBUF * g
            for b in range(NBUF):
                drains[b].wait()
                pltpu.sync_copy(rows[b], acc_sh.at[dus[b].at[0]], add=True)
                unpack(j + NBUF + b, sus[b], dus[b])
                pltpu.async_copy(h_hbm.at[sus[b].at[0]], rows[b], sems[b])
            return 0

        lax.fori_loop(0, nchunks // NBUF - 1, body, 0)
        for b in range(NBUF):
            drains[b].wait()
            pltpu.sync_copy(rows[b], acc_sh.at[dus[b].at[0]], add=True)
        plsc.subcore_barrier()

        def write_rows(nrows):
            ob = sid * opart
            pltpu.sync_copy(acc_sh.at[pl.ds(ob, nrows)],
                            out_hbm.at[cid].at[pl.ds(ob, nrows)])

        pl.when(sid < ns - 1)(lambda: write_rows(opart))
        pl.when(sid == ns - 1)(lambda: write_rows(olast))

    return k(h, pk3)


def _tc_self_body(x_ref, ws_ref, o_ref):
    o_ref[...] = jax.lax.dot_general(
        x_ref[...], ws_ref[...], (((1,), (0,)), ((), ())),
        preferred_element_type=jnp.float32, precision=lax.Precision.HIGHEST)


@functools.partial(jax.jit, static_argnames=("br",))
def _tc_self(x, ws, *, br):
    n, d = x.shape
    return pl.pallas_call(
        _tc_self_body,
        grid=(n // br,),
        in_specs=[pl.BlockSpec((br, d), lambda i: (i, 0)),
                  pl.BlockSpec((d, d), lambda i: (0, 0))],
        out_specs=pl.BlockSpec((br, d), lambda i: (i, 0)),
        out_shape=jax.ShapeDtypeStruct((n, d), jnp.float32),
    )(x, ws)


def _tc_combA_body(s_ref, p0_ref, p1_ref, deg_ref, wn_ref, b_ref,
                   g_ref, lb_ref, ws1_ref, h8_ref, self1_ref, hscr):
    @pl.when(pl.program_id(1) == 0)
    def _():
        agg = (p0_ref[...] + p1_ref[...]) / jnp.clip(deg_ref[...], 1.0, None)
        h = (
            s_ref[...]
            + jax.lax.dot_general(
                agg, wn_ref[...], (((1,), (0,)), ((), ())),
                preferred_element_type=jnp.float32,
                precision=lax.Precision.HIGHEST)
            + b_ref[...]
        )
        mu = jnp.mean(h, axis=-1, keepdims=True)
        var = jnp.mean(jnp.square(h - mu), axis=-1, keepdims=True)
        h = (h - mu) / jnp.sqrt(var + 1e-5) * g_ref[...] + lb_ref[...]
        h = jnp.maximum(h, 0.0)
        hscr[...] = h
        self1_ref[...] = jax.lax.dot_general(
            h, ws1_ref[...], (((1,), (0,)), ((), ())),
            preferred_element_type=jnp.float32, precision=lax.Precision.HIGHEST)
    h8_ref[0] = hscr[...]


@functools.partial(jax.jit, static_argnames=("br", "reps"))
def _tc_combineA(s, p0, p1, deg2, wn, b, g, lb, ws1, *, br, reps):
    n, d = s.shape
    row_spec = pl.BlockSpec((br, d), lambda i, r: (i, 0))
    deg_spec = pl.BlockSpec((br, 1), lambda i, r: (i, 0))
    w_spec = pl.BlockSpec((d, d), lambda i, r: (0, 0))
    v_spec = pl.BlockSpec((1, d), lambda i, r: (0, 0))
    return pl.pallas_call(
        _tc_combA_body,
        grid=(n // br, reps),
        in_specs=[row_spec, row_spec, row_spec, deg_spec,
                  w_spec, v_spec, v_spec, v_spec, w_spec],
        out_specs=[pl.BlockSpec((1, br, d), lambda i, r: (r, i, 0)),
                   row_spec],
        out_shape=[jax.ShapeDtypeStruct((reps, n, d), jnp.float32),
                   jax.ShapeDtypeStruct((n, d), jnp.float32)],
        scratch_shapes=[pltpu.VMEM((br, d), jnp.float32)],
    )(s, p0, p1, deg2, wn, b, g, lb, ws1)


def _tc_comb_body(s_ref, p0_ref, p1_ref, deg_ref, wn_ref, b_ref,
                  g_ref, lb_ref, o_ref, *, ln_relu):
    agg = (p0_ref[...] + p1_ref[...]) / jnp.clip(deg_ref[...], 1.0, None)
    h = (
        s_ref[...]
        + jax.lax.dot_general(
            agg, wn_ref[...], (((1,), (0,)), ((), ())),
            preferred_element_type=jnp.float32, precision=lax.Precision.HIGHEST)
        + b_ref[...]
    )
    if ln_relu:
        mu = jnp.mean(h, axis=-1, keepdims=True)
        var = jnp.mean(jnp.square(h - mu), axis=-1, keepdims=True)
        h = (h - mu) / jnp.sqrt(var + 1e-5) * g_ref[...] + lb_ref[...]
        h = jnp.maximum(h, 0.0)
    o_ref[...] = h


@functools.partial(jax.jit, static_argnames=("ln_relu", "br"))
def _tc_combine(s, p0, p1, deg2, wn, b, g, lb, *, ln_relu, br):
    n, d = s.shape
    row_spec = pl.BlockSpec((br, d), lambda i: (i, 0))
    deg_spec = pl.BlockSpec((br, 1), lambda i: (i, 0))
    w_spec = pl.BlockSpec((d, d), lambda i: (0, 0))
    v_spec = pl.BlockSpec((1, d), lambda i: (0, 0))
    return pl.pallas_call(
        functools.partial(_tc_comb_body, ln_relu=ln_relu),
        grid=(n // br,),
        in_specs=[row_spec, row_spec, row_spec, deg_spec,
                  w_spec, v_spec, v_spec, v_spec],
        out_specs=row_spec,
        out_shape=jax.ShapeDtypeStruct((n, d), jnp.float32),
    )(s, p0, p1, deg2, wn, b, g, lb)


def kernel(feat, edge_index, in_deg, W_self0, W_neigh0, b0,
           W_self1, W_neigh1, b1, ln_g, ln_b):
    n, d = feat.shape
    e = edge_index.shape[1]
    nc, ns = 2, 16
    nw = nc * ns
    per_w = ((e + nw * NBUF * CHUNK - 1) // (nw * NBUF * CHUNK)) * NBUF * CHUNK
    nchunks = per_w // CHUNK
    e_pad = per_w * nw

    src = edge_index[0]
    dst = edge_index[1]
    rw = (e + nw - 1) // nw
    padw = per_w - rw
    n_dummy = max(padw + (nw * rw - e), 1)
    n_acc = ((n + n_dummy + 7) // 8) * 8
    pk = src | (dst << 16)
    pk = jnp.concatenate(
        [pk, ((n + jnp.arange(nw * rw - e, dtype=jnp.int32)) << 16)])
    pk = pk.reshape(nw, rw)
    pad_blk = ((n + (nw * rw - e)
                + jnp.arange(padw, dtype=jnp.int32)) << 16)
    pk3 = jnp.concatenate(
        [pk, jnp.broadcast_to(pad_blk, (nw, padw))], axis=1
    ).reshape(nw, nchunks // 4, 4 * CHUNK)
    deg2 = in_deg.reshape(n, 1)
    b0r, b1r = b0.reshape(1, d), b1.reshape(1, d)
    gr, lbr = ln_g.reshape(1, d), ln_b.reshape(1, d)

    br = 1000 if n % 1000 == 0 else 8 * (n // 8)

    feat8 = jnp.concatenate([feat] * 8, axis=0)
    p = _sc_aggregate(feat8, pk3, n=n, d=d, nc=nc, ns=ns, nchunks=nchunks,
                      n_acc=n_acc)
    self0 = _tc_self(feat, W_self0, br=br)
    h1x8, self1 = _tc_combineA(self0, p[0], p[1], deg2, W_neigh0, b0r, gr,
                               lbr, W_self1, br=br, reps=8)
    p = _sc_aggregate(h1x8.reshape(8 * n, d), pk3, n=n, d=d, nc=nc, ns=ns,
                      nchunks=nchunks, n_acc=n_acc)
    out = _tc_combine(self1, p[0], p[1], deg2, W_neigh1, b1r, gr, lbr,
                      ln_relu=False, br=br)
    return out


# speedup vs baseline: 1.1156x; 1.1156x over previous
import functools

import jax
import jax.numpy as jnp
from jax import lax
from jax.experimental import pallas as pl
from jax.experimental.pallas import tpu as pltpu
from jax.experimental.pallas import tpu_sc as plsc

CHUNK = 32
NBUF = 8


@functools.partial(
    jax.jit, static_argnames=("n", "d", "nc", "ns", "nchunks", "n_acc"))
def _sc_aggregate(h, pk3, *, n, d, nc, ns, nchunks, n_acc):
    zpart = 8 * (n_acc // (8 * ns))
    zlast = n_acc - zpart * (ns - 1)
    opart = 8 * (n // (8 * ns))
    olast = n - opart * (ns - 1)
    mesh = plsc.VectorSubcoreMesh(core_axis_name="c", subcore_axis_name="s")

    @functools.partial(
        pl.kernel,
        out_type=jax.ShapeDtypeStruct((nc, n, d), jnp.float32),
        mesh=mesh,
        scratch_types=[
            pltpu.VMEM((nchunks // 4, 4 * CHUNK), jnp.int32),
            *([pltpu.VMEM((1, CHUNK), jnp.int32)] * (2 * NBUF)),
            *([pltpu.VMEM((CHUNK, d), jnp.float32)] * NBUF),
            pltpu.VMEM_SHARED((n_acc, d), jnp.float32),
            *([pltpu.SemaphoreType.DMA] * NBUF),
        ],
    )
    def k(h_hbm, pk_hbm, out_hbm, pk_v, *rest):
        sus = rest[0:2 * NBUF:2]
        dus = rest[1:2 * NBUF:2]
        rows = rest[2 * NBUF:3 * NBUF]
        acc_sh = rest[3 * NBUF]
        sems = rest[3 * NBUF + 1:]
        rows_v = rows[0]
        cid = lax.axis_index("c")
        sid = lax.axis_index("s")
        wid = cid * ns + sid

        def zrow(i, _):
            for k8 in range(d // 16):
                rows_v[i, pl.ds(k8 * 16, 16)] = jnp.zeros((16,), jnp.float32)
            return 0

        lax.fori_loop(0, CHUNK, zrow, 0)

        def zero_rows(nrows):
            base = sid * zpart
            full, rem = nrows // CHUNK, nrows % CHUNK
            for z in range(full):
                pltpu.sync_copy(rows_v, acc_sh.at[pl.ds(base + z * CHUNK, CHUNK)])
            if rem:
                pltpu.sync_copy(rows_v.at[pl.ds(0, rem)],
                                acc_sh.at[pl.ds(base + full * CHUNK, rem)])

        pl.when(sid < ns - 1)(lambda: zero_rows(zpart))
        pl.when(sid == ns - 1)(lambda: zero_rows(zlast))
        plsc.subcore_barrier()

        pltpu.sync_copy(pk_hbm.at[wid], pk_v)

        hoff = (cid * 4 + lax.div(sid, 4)) * n

        def unpack(j, su, du):
            half = (j % 4) * CHUNK
            for k8 in range(CHUNK // 16):
                w = pk_v[j // 4, pl.ds(half + k8 * 16, 16)]
                su[0, pl.ds(k8 * 16, 16)] = (w & 0xFFFF) + hoff
                du[0, pl.ds(k8 * 16, 16)] = lax.shift_right_logical(w, 16)

        drains = [pltpu.make_async_copy(h_hbm.at[sus[b].at[0]], rows[b], sems[b])
                  for b in range(NBUF)]
        for b in range(NBUF):
            unpack(b, sus[b], dus[b])
            pltpu.async_copy(h_hbm.at[sus[b].at[0]], rows[b], sems[b])

        def body(g, _):
            j = NBUF * g
            for b in range(NBUF):
                drains[b].wait()
                pltpu.sync_copy(rows[b], acc_sh.at[dus[b].at[0]], add=True)
                unpack(j + NBUF + b, sus[b], dus[b])
                pltpu.async_copy(h_hbm.at[sus[b].at[0]], rows[b], sems[b])
            return 0

        lax.fori_loop(0, nchunks // NBUF - 1, body, 0)
        for b in range(NBUF):
            drains[b].wait()
            pltpu.sync_copy(rows[b], acc_sh.at[dus[b].at[0]], add=True)
        plsc.subcore_barrier()

        def write_rows(nrows):
            ob = sid * opart
            pltpu.sync_copy(acc_sh.at[pl.ds(ob, nrows)],
                            out_hbm.at[cid].at[pl.ds(ob, nrows)])

        pl.when(sid < ns - 1)(lambda: write_rows(opart))
        pl.when(sid == ns - 1)(lambda: write_rows(olast))

    return k(h, pk3)


def _tc_self_body(x_ref, ws_ref, o_ref):
    o_ref[...] = jax.lax.dot_general(
        x_ref[...], ws_ref[...], (((1,), (0,)), ((), ())),
        preferred_element_type=jnp.float32, precision=lax.Precision.HIGHEST)


@functools.partial(jax.jit, static_argnames=("br",))
def _tc_self(x, ws, *, br):
    n, d = x.shape
    return pl.pallas_call(
        _tc_self_body,
        grid=(n // br,),
        in_specs=[pl.BlockSpec((br, d), lambda i: (i, 0)),
                  pl.BlockSpec((d, d), lambda i: (0, 0))],
        out_specs=pl.BlockSpec((br, d), lambda i: (i, 0)),
        out_shape=jax.ShapeDtypeStruct((n, d), jnp.float32),
    )(x, ws)


def _tc_comb_body(s_ref, p0_ref, p1_ref, deg_ref, wn_ref, b_ref,
                  g_ref, lb_ref, o_ref, *, ln_relu):
    agg = (p0_ref[...] + p1_ref[...]) / jnp.clip(deg_ref[...], 1.0, None)
    h = (
        s_ref[...]
        + jax.lax.dot_general(
            agg, wn_ref[...], (((1,), (0,)), ((), ())),
            preferred_element_type=jnp.float32, precision=lax.Precision.HIGHEST)
        + b_ref[...]
    )
    if ln_relu:
        mu = jnp.mean(h, axis=-1, keepdims=True)
        var = jnp.mean(jnp.square(h - mu), axis=-1, keepdims=True)
        h = (h - mu) / jnp.sqrt(var + 1e-5) * g_ref[...] + lb_ref[...]
        h = jnp.maximum(h, 0.0)
    o_ref[...] = h


@functools.partial(jax.jit, static_argnames=("ln_relu", "br"))
def _tc_combine(s, p0, p1, deg2, wn, b, g, lb, *, ln_relu, br):
    n, d = s.shape
    row_spec = pl.BlockSpec((br, d), lambda i: (i, 0))
    deg_spec = pl.BlockSpec((br, 1), lambda i: (i, 0))
    w_spec = pl.BlockSpec((d, d), lambda i: (0, 0))
    v_spec = pl.BlockSpec((1, d), lambda i: (0, 0))
    return pl.pallas_call(
        functools.partial(_tc_comb_body, ln_relu=ln_relu),
        grid=(n // br,),
        in_specs=[row_spec, row_spec, row_spec, deg_spec,
                  w_spec, v_spec, v_spec, v_spec],
        out_specs=row_spec,
        out_shape=jax.ShapeDtypeStruct((n, d), jnp.float32),
    )(s, p0, p1, deg2, wn, b, g, lb)


def kernel(feat, edge_index, in_deg, W_self0, W_neigh0, b0,
           W_self1, W_neigh1, b1, ln_g, ln_b):
    n, d = feat.shape
    e = edge_index.shape[1]
    nc, ns = 2, 16
    nw = nc * ns
    per_w = ((e + nw * NBUF * CHUNK - 1) // (nw * NBUF * CHUNK)) * NBUF * CHUNK
    nchunks = per_w // CHUNK
    e_pad = per_w * nw

    src = edge_index[0]
    dst = edge_index[1]
    rw = (e + nw - 1) // nw
    padw = per_w - rw
    n_dummy = max(padw + (nw * rw - e), 1)
    n_acc = ((n + n_dummy + 7) // 8) * 8
    pk = src | (dst << 16)
    pk = jnp.concatenate(
        [pk, ((n + jnp.arange(nw * rw - e, dtype=jnp.int32)) << 16)])
    pk = pk.reshape(nw, rw)
    pad_blk = ((n + (nw * rw - e)
                + jnp.arange(padw, dtype=jnp.int32)) << 16)
    pk3 = jnp.concatenate(
        [pk, jnp.broadcast_to(pad_blk, (nw, padw))], axis=1
    ).reshape(nw, nchunks // 4, 4 * CHUNK)
    deg2 = in_deg.reshape(n, 1)
    b0r, b1r = b0.reshape(1, d), b1.reshape(1, d)
    gr, lbr = ln_g.reshape(1, d), ln_b.reshape(1, d)

    br = 1000 if n % 1000 == 0 else 8 * (n // 8)

    feat8 = jnp.concatenate([feat] * 8, axis=0)
    p = _sc_aggregate(feat8, pk3, n=n, d=d, nc=nc, ns=ns, nchunks=nchunks,
                      n_acc=n_acc)
    self0 = _tc_self(feat, W_self0, br=br)
    h1 = _tc_combine(self0, p[0], p[1], deg2, W_neigh0, b0r, gr, lbr,
                     ln_relu=True, br=br)
    h1x8 = jnp.concatenate([h1] * 8, axis=0)
    p = _sc_aggregate(h1x8, pk3, n=n, d=d, nc=nc, ns=ns, nchunks=nchunks,
                      n_acc=n_acc)
    self1 = _tc_self(h1, W_self1, br=br)
    out = _tc_combine(self1, p[0], p[1], deg2, W_neigh1, b1r, gr, lbr,
                      ln_relu=False, br=br)
    return out


# speedup vs baseline: 1.1386x; 1.0206x over previous
import functools

import jax
import jax.numpy as jnp
from jax import lax
from jax.experimental import pallas as pl
from jax.experimental.pallas import tpu as pltpu
from jax.experimental.pallas import tpu_sc as plsc

CHUNK = 32
NBUF = 8


@functools.partial(
    jax.jit, static_argnames=("n", "d", "nc", "ns", "nchunks", "n_acc"))
def _sc_aggregate(h, pk3, *, n, d, nc, ns, nchunks, n_acc):
    zpart = 8 * (n_acc // (8 * ns))
    zlast = n_acc - zpart * (ns - 1)
    opart = 8 * (n // (8 * ns))
    olast = n - opart * (ns - 1)
    mesh = plsc.VectorSubcoreMesh(core_axis_name="c", subcore_axis_name="s")

    @functools.partial(
        pl.kernel,
        out_type=jax.ShapeDtypeStruct((nc, n, d), jnp.float32),
        mesh=mesh,
        scratch_types=[
            pltpu.VMEM((nchunks // 4, 4 * CHUNK), jnp.int32),
            *([pltpu.VMEM((1, CHUNK), jnp.int32)] * (2 * NBUF)),
            *([pltpu.VMEM((CHUNK, d), jnp.float32)] * NBUF),
            pltpu.VMEM_SHARED((n_acc, d), jnp.float32),
            *([pltpu.SemaphoreType.DMA] * NBUF),
        ],
    )
    def k(h_hbm, pk_hbm, out_hbm, pk_v, *rest):
        sus = rest[0:2 * NBUF:2]
        dus = rest[1:2 * NBUF:2]
        rows = rest[2 * NBUF:3 * NBUF]
        acc_sh = rest[3 * NBUF]
        sems = rest[3 * NBUF + 1:]
        rows_v = rows[0]
        cid = lax.axis_index("c")
        sid = lax.axis_index("s")
        wid = cid * ns + sid

        def zrow(i, _):
            for k8 in range(d // 16):
                rows_v[i, pl.ds(k8 * 16, 16)] = jnp.zeros((16,), jnp.float32)
            return 0

        lax.fori_loop(0, CHUNK, zrow, 0)

        def zero_rows(nrows):
            base = sid * zpart
            full, rem = nrows // CHUNK, nrows % CHUNK
            for z in range(full):
                pltpu.sync_copy(rows_v, acc_sh.at[pl.ds(base + z * CHUNK, CHUNK)])
            if rem:
                pltpu.sync_copy(rows_v.at[pl.ds(0, rem)],
                                acc_sh.at[pl.ds(base + full * CHUNK, rem)])

        pl.when(sid < ns - 1)(lambda: zero_rows(zpart))
        pl.when(sid == ns - 1)(lambda: zero_rows(zlast))
        plsc.subcore_barrier()

        pltpu.sync_copy(pk_hbm.at[wid], pk_v)

        hoff = (cid * 4 + lax.div(sid, 4)) * n

        def unpack(j, su, du):
            half = (j % 4) * CHUNK
            for k8 in range(CHUNK // 16):
                w = pk_v[j // 4, pl.ds(half + k8 * 16, 16)]
                su[0, pl.ds(k8 * 16, 16)] = (w & 0xFFFF) + hoff
                du[0, pl.ds(k8 * 16, 16)] = lax.shift_right_logical(w, 16)

        drains = [pltpu.make_async_copy(h_hbm.at[sus[b].at[0]], rows[b], sems[b])
                  for b in range(NBUF)]
        for b in range(NBUF):
            unpack(b, sus[b], dus[b])
            pltpu.async_copy(h_hbm.at[sus[b].at[0]], rows[b], sems[b])

        def body(g, _):
            j = NBUF * g
            for b in range(NBUF):
                drains[b].wait()
                pltpu.sync_copy(rows[b], acc_sh.at[dus[b].at[0]], add=True)
                unpack(j + NBUF + b, sus[b], dus[b])
                pltpu.async_copy(h_hbm.at[sus[b].at[0]], rows[b], sems[b])
            return 0

        lax.fori_loop(0, nchunks // NBUF - 1, body, 0)
        for b in range(NBUF):
            drains[b].wait()
            pltpu.sync_copy(rows[b], acc_sh.at[dus[b].at[0]], add=True)
        plsc.subcore_barrier()

        def write_rows(nrows):
            ob = sid * opart
            pltpu.sync_copy(acc_sh.at[pl.ds(ob, nrows)],
                            out_hbm.at[cid].at[pl.ds(ob, nrows)])

        pl.when(sid < ns - 1)(lambda: write_rows(opart))
        pl.when(sid == ns - 1)(lambda: write_rows(olast))

    return k(h, pk3)


def _tc_self_body(x_ref, ws_ref, o_ref):
    o_ref[...] = jax.lax.dot_general(
        x_ref[...], ws_ref[...], (((1,), (0,)), ((), ())),
        preferred_element_type=jnp.float32, precision=lax.Precision.HIGHEST)


@functools.partial(jax.jit, static_argnames=("br",))
def _tc_self(x, ws, *, br):
    n, d = x.shape
    return pl.pallas_call(
        _tc_self_body,
        grid=(n // br,),
        in_specs=[pl.BlockSpec((br, d), lambda i: (i, 0)),
                  pl.BlockSpec((d, d), lambda i: (0, 0))],
        out_specs=pl.BlockSpec((br, d), lambda i: (i, 0)),
        out_shape=jax.ShapeDtypeStruct((n, d), jnp.float32),
    )(x, ws)


def _tc_comb_body(s_ref, p0_ref, p1_ref, deg_ref, wn_ref, b_ref,
                  g_ref, lb_ref, o_ref, *, ln_relu):
    agg = (p0_ref[...] + p1_ref[...]) / jnp.clip(deg_ref[...], 1.0, None)
    h = (
        s_ref[...]
        + jax.lax.dot_general(
            agg, wn_ref[...], (((1,), (0,)), ((), ())),
            preferred_element_type=jnp.float32, precision=lax.Precision.HIGHEST)
        + b_ref[...]
    )
    if ln_relu:
        mu = jnp.mean(h, axis=-1, keepdims=True)
        var = jnp.mean(jnp.square(h - mu), axis=-1, keepdims=True)
        h = (h - mu) / jnp.sqrt(var + 1e-5) * g_ref[...] + lb_ref[...]
        h = jnp.maximum(h, 0.0)
    o_ref[...] = h


@functools.partial(jax.jit, static_argnames=("ln_relu", "br"))
def _tc_combine(s, p0, p1, deg2, wn, b, g, lb, *, ln_relu, br):
    n, d = s.shape
    row_spec = pl.BlockSpec((br, d), lambda i: (i, 0))
    deg_spec = pl.BlockSpec((br, 1), lambda i: (i, 0))
    w_spec = pl.BlockSpec((d, d), lambda i: (0, 0))
    v_spec = pl.BlockSpec((1, d), lambda i: (0, 0))
    return pl.pallas_call(
        functools.partial(_tc_comb_body, ln_relu=ln_relu),
        grid=(n // br,),
        in_specs=[row_spec, row_spec, row_spec, deg_spec,
                  w_spec, v_spec, v_spec, v_spec],
        out_specs=row_spec,
        out_shape=jax.ShapeDtypeStruct((n, d), jnp.float32),
    )(s, p0, p1, deg2, wn, b, g, lb)


def kernel(feat, edge_index, in_deg, W_self0, W_neigh0, b0,
           W_self1, W_neigh1, b1, ln_g, ln_b):
    n, d = feat.shape
    e = edge_index.shape[1]
    nc, ns = 2, 16
    nw = nc * ns
    per_w = ((e + nw * NBUF * CHUNK - 1) // (nw * NBUF * CHUNK)) * NBUF * CHUNK
    nchunks = per_w // CHUNK
    e_pad = per_w * nw

    src = edge_index[0]
    dst = edge_index[1]
    rw = (e + nw - 1) // nw
    padw = per_w - rw
    n_dummy = max(padw + (nw * rw - e), 1)
    n_acc = ((n + n_dummy + 7) // 8) * 8
    pk = src | (dst << 16)
    pk = jnp.concatenate(
        [pk, ((n + jnp.arange(nw * rw - e, dtype=jnp.int32)) << 16)])
    pk = pk.reshape(nw, rw)
    pad_blk = ((n + (nw * rw - e)
                + jnp.arange(padw, dtype=jnp.int32)) << 16)
    pk3 = jnp.concatenate(
        [pk, jnp.broadcast_to(pad_blk, (nw, padw))], axis=1
    ).reshape(nw, nchunks // 4, 4 * CHUNK)
    deg2 = in_deg.reshape(n, 1)
    b0r, b1r = b0.reshape(1, d), b1.reshape(1, d)
    gr, lbr = ln_g.reshape(1, d), ln_b.reshape(1, d)

    br = 2000 if n % 2000 == 0 else 8 * (n // 8)

    feat8 = jnp.concatenate([feat] * 8, axis=0)
    p = _sc_aggregate(feat8, pk3, n=n, d=d, nc=nc, ns=ns, nchunks=nchunks,
                      n_acc=n_acc)
    self0 = _tc_self(feat, W_self0, br=br)
    h1 = _tc_combine(self0, p[0], p[1], deg2, W_neigh0, b0r, gr, lbr,
                     ln_relu=True, br=br)
    h1x8 = jnp.concatenate([h1] * 8, axis=0)
    p = _sc_aggregate(h1x8, pk3, n=n, d=d, nc=nc, ns=ns, nchunks=nchunks,
                      n_acc=n_acc)
    self1 = _tc_self(h1, W_self1, br=br)
    out = _tc_combine(self1, p[0], p[1], deg2, W_neigh1, b1r, gr, lbr,
                      ln_relu=False, br=br)
    return out
